# baseline (speedup 1.0000x reference)
import functools

import jax
import jax.numpy as jnp
from jax.experimental import pallas as pl

_CHUNK = 256
_DOT = functools.partial(
    jax.lax.dot_general,
    dimension_numbers=(((1,), (0,)), ((), ())),
    preferred_element_type=jnp.float32,
)


def _cumprod_block(x_ref, u_ref, o_ref):
    x = x_ref[...]
    rows, n = x.shape
    u = u_ref[...]
    l = jnp.maximum(jnp.log2(jnp.abs(x)), jnp.float32(-1e30))
    neg = jnp.where(x < 0, 1.0, 0.0).astype(jnp.bfloat16)
    hi = l.astype(jnp.bfloat16)
    lo = (l - hi.astype(jnp.float32)).astype(jnp.bfloat16)
    s_parts = []
    c_parts = []
    for c in range(n // _CHUNK):
        sl = slice(c * _CHUNK, (c + 1) * _CHUNK)
        s_parts.append(_DOT(hi[:, sl], u) + _DOT(lo[:, sl], u))
        c_parts.append(_DOT(neg[:, sl], u))
    for c in range(1, n // _CHUNK):
        s_parts[c] = s_parts[c] + jax.lax.broadcast_in_dim(
            s_parts[c - 1][:, _CHUNK - 1], (rows, _CHUNK), (0,))
        c_parts[c] = c_parts[c] + jax.lax.broadcast_in_dim(
            c_parts[c - 1][:, _CHUNK - 1], (rows, _CHUNK), (0,))
    s = jnp.concatenate(s_parts, axis=1)
    cnt = jnp.concatenate(c_parts, axis=1)
    bit = jnp.left_shift(jnp.bitwise_and(cnt.astype(jnp.int32), 1), 31)
    e_bits = jax.lax.bitcast_convert_type(jnp.exp2(s), jnp.int32)
    o_ref[...] = jax.lax.bitcast_convert_type(e_bits | bit, jnp.float32)


def kernel(x):
    m, n = x.shape
    block_rows = 256
    u = jnp.triu(jnp.ones((_CHUNK, _CHUNK), jnp.bfloat16))
    return pl.pallas_call(
        _cumprod_block,
        out_shape=jax.ShapeDtypeStruct((m, n), x.dtype),
        grid=(m // block_rows,),
        in_specs=[
            pl.BlockSpec((block_rows, n), lambda i: (i, 0)),
            pl.BlockSpec((_CHUNK, _CHUNK), lambda i: (0, 0)),
        ],
        out_specs=pl.BlockSpec((block_rows, n), lambda i: (i, 0)),
    )(x, u)


# speedup vs baseline: 1.4821x; 1.4821x over previous
import functools

import jax
import jax.numpy as jnp
from jax.experimental import pallas as pl

_CHUNK = 256
_DOT = functools.partial(
    jax.lax.dot_general,
    dimension_numbers=(((1,), (0,)), ((), ())),
    preferred_element_type=jnp.float32,
)


def _cumprod_block(x_ref, u_ref, o_ref):
    x = x_ref[...]
    rows, n = x.shape
    u = u_ref[...]
    l = jnp.maximum(jnp.log2(jnp.abs(x)), jnp.float32(-1e30))
    neg = jnp.where(x < 0, 1.0, 0.0).astype(jnp.bfloat16)
    hi = l.astype(jnp.bfloat16)
    lo = (l - hi.astype(jnp.float32)).astype(jnp.bfloat16)
    s_parts = []
    c_parts = []
    for c in range(n // _CHUNK):
        sl = slice(c * _CHUNK, (c + 1) * _CHUNK)
        s_parts.append(_DOT(hi[:, sl], u) + _DOT(lo[:, sl], u))
        c_parts.append(_DOT(neg[:, sl], u))
    for c in range(1, n // _CHUNK):
        s_parts[c] = s_parts[c] + jax.lax.broadcast_in_dim(
            s_parts[c - 1][:, _CHUNK - 1], (rows, _CHUNK), (0,))
        c_parts[c] = c_parts[c] + jax.lax.broadcast_in_dim(
            c_parts[c - 1][:, _CHUNK - 1], (rows, _CHUNK), (0,))
    s = jnp.concatenate(s_parts, axis=1)
    cnt = jnp.concatenate(c_parts, axis=1)
    bit = jnp.left_shift(jnp.bitwise_and(cnt.astype(jnp.int32), 1), 31)
    e_bits = jax.lax.bitcast_convert_type(jnp.exp2(s), jnp.int32)
    o_ref[...] = jax.lax.bitcast_convert_type(e_bits | bit, jnp.float32)


def kernel(x):
    m, n = x.shape
    block_rows = 1024
    u = jnp.triu(jnp.ones((_CHUNK, _CHUNK), jnp.bfloat16))
    return pl.pallas_call(
        _cumprod_block,
        out_shape=jax.ShapeDtypeStruct((m, n), x.dtype),
        grid=(m // block_rows,),
        in_specs=[
            pl.BlockSpec((block_rows, n), lambda i: (i, 0)),
            pl.BlockSpec((_CHUNK, _CHUNK), lambda i: (0, 0)),
        ],
        out_specs=pl.BlockSpec((block_rows, n), lambda i: (i, 0)),
    )(x, u)


# speedup vs baseline: 1.5504x; 1.0460x over previous
import functools

import jax
import jax.numpy as jnp
from jax.experimental import pallas as pl

_CHUNK = 256
_DOT = functools.partial(
    jax.lax.dot_general,
    dimension_numbers=(((1,), (0,)), ((), ())),
    preferred_element_type=jnp.float32,
)


def _cumprod_block(x_ref, u_ref, o_ref):
    x = x_ref[...]
    rows, n = x.shape
    u = u_ref[...]
    l = jnp.maximum(jnp.log2(jnp.abs(x)), jnp.float32(-1e30))
    neg = jnp.where(x < 0, 1.0, 0.0).astype(jnp.bfloat16)
    hi = l.astype(jnp.bfloat16)
    lo = (l - hi.astype(jnp.float32)).astype(jnp.bfloat16)
    s_parts = []
    c_parts = []
    for c in range(n // _CHUNK):
        sl = slice(c * _CHUNK, (c + 1) * _CHUNK)
        s_parts.append(_DOT(hi[:, sl], u) + _DOT(lo[:, sl], u))
        c_parts.append(_DOT(neg[:, sl], u))
    for c in range(1, n // _CHUNK):
        s_parts[c] = s_parts[c] + jax.lax.broadcast_in_dim(
            s_parts[c - 1][:, _CHUNK - 1], (rows, _CHUNK), (0,))
        c_parts[c] = c_parts[c] + jax.lax.broadcast_in_dim(
            c_parts[c - 1][:, _CHUNK - 1], (rows, _CHUNK), (0,))
    s = jnp.concatenate(s_parts, axis=1)
    cnt = jnp.concatenate(c_parts, axis=1)
    bit = jnp.left_shift(jnp.bitwise_and(cnt.astype(jnp.int32), 1), 31)
    e_bits = jax.lax.bitcast_convert_type(jnp.exp2(s), jnp.int32)
    o_ref[...] = jax.lax.bitcast_convert_type(e_bits | bit, jnp.float32)


def kernel(x):
    m, n = x.shape
    block_rows = 2048
    u = jnp.triu(jnp.ones((_CHUNK, _CHUNK), jnp.bfloat16))
    return pl.pallas_call(
        _cumprod_block,
        out_shape=jax.ShapeDtypeStruct((m, n), x.dtype),
        grid=(m // block_rows,),
        in_specs=[
            pl.BlockSpec((block_rows, n), lambda i: (i, 0)),
            pl.BlockSpec((_CHUNK, _CHUNK), lambda i: (0, 0)),
        ],
        out_specs=pl.BlockSpec((block_rows, n), lambda i: (i, 0)),
    )(x, u)


# speedup vs baseline: 1.5606x; 1.0066x over previous
import functools

import jax
import jax.numpy as jnp
from jax.experimental import pallas as pl

_CHUNK = 256
_DOT = functools.partial(
    jax.lax.dot_general,
    dimension_numbers=(((1,), (0,)), ((), ())),
    preferred_element_type=jnp.float32,
)


def _cumprod_block(x_ref, u_ref, o_ref):
    x = x_ref[...]
    rows, n = x.shape
    u = u_ref[...]
    l = jnp.maximum(jnp.log2(jnp.abs(x)), jnp.float32(-1e30))
    neg = jnp.where(x < 0, 1.0, 0.0).astype(jnp.bfloat16)
    hi_f = jax.lax.bitcast_convert_type(
        jnp.bitwise_and(jax.lax.bitcast_convert_type(l, jnp.int32),
                        jnp.int32(-65536)), jnp.float32)
    hi = hi_f.astype(jnp.bfloat16)
    lo = (l - hi_f).astype(jnp.bfloat16)
    s_parts = []
    c_parts = []
    for c in range(n // _CHUNK):
        sl = slice(c * _CHUNK, (c + 1) * _CHUNK)
        s_parts.append(_DOT(hi[:, sl], u) + _DOT(lo[:, sl], u))
        c_parts.append(_DOT(neg[:, sl], u))
    for c in range(1, n // _CHUNK):
        s_parts[c] = s_parts[c] + jax.lax.broadcast_in_dim(
            s_parts[c - 1][:, _CHUNK - 1], (rows, _CHUNK), (0,))
        c_parts[c] = c_parts[c] + jax.lax.broadcast_in_dim(
            c_parts[c - 1][:, _CHUNK - 1], (rows, _CHUNK), (0,))
    s = jnp.concatenate(s_parts, axis=1)
    cnt = jnp.concatenate(c_parts, axis=1)
    bit = jnp.left_shift(jnp.bitwise_and(
        jax.lax.bitcast_convert_type(cnt + jnp.float32(8388608.0), jnp.int32),
        1), 31)
    e_bits = jax.lax.bitcast_convert_type(jnp.exp2(s), jnp.int32)
    o_ref[...] = jax.lax.bitcast_convert_type(e_bits | bit, jnp.float32)


def kernel(x):
    m, n = x.shape
    block_rows = 2048
    u = jnp.triu(jnp.ones((_CHUNK, _CHUNK), jnp.bfloat16))
    return pl.pallas_call(
        _cumprod_block,
        out_shape=jax.ShapeDtypeStruct((m, n), x.dtype),
        grid=(m // block_rows,),
        in_specs=[
            pl.BlockSpec((block_rows, n), lambda i: (i, 0)),
            pl.BlockSpec((_CHUNK, _CHUNK), lambda i: (0, 0)),
        ],
        out_specs=pl.BlockSpec((block_rows, n), lambda i: (i, 0)),
    )(x, u)


# speedup vs baseline: 1.5852x; 1.0157x over previous
import functools

import jax
import jax.numpy as jnp
from jax.experimental import pallas as pl

_CHUNK = 256
_DOT = functools.partial(
    jax.lax.dot_general,
    dimension_numbers=(((1,), (0,)), ((), ())),
    preferred_element_type=jnp.float32,
)


def _cumprod_block(x_ref, u_ref, o_ref):
    rows, n = x_ref.shape
    u = u_ref[...]
    s_carry = None
    c_carry = None
    for c in range(n // _CHUNK):
        sl = slice(c * _CHUNK, (c + 1) * _CHUNK)
        xc = x_ref[:, sl]
        l = jnp.maximum(jnp.log2(jnp.abs(xc)), jnp.float32(-1e30))
        neg = jnp.where(xc < 0, 1.0, 0.0).astype(jnp.bfloat16)
        hi_f = jax.lax.bitcast_convert_type(
            jnp.bitwise_and(jax.lax.bitcast_convert_type(l, jnp.int32),
                            jnp.int32(-65536)), jnp.float32)
        hi = hi_f.astype(jnp.bfloat16)
        lo = (l - hi_f).astype(jnp.bfloat16)
        s = _DOT(hi, u) + _DOT(lo, u)
        cnt = _DOT(neg, u)
        if c:
            s = s + jax.lax.broadcast_in_dim(s_carry, (rows, _CHUNK), (0,))
            cnt = cnt + jax.lax.broadcast_in_dim(c_carry, (rows, _CHUNK), (0,))
        s_carry = s[:, _CHUNK - 1]
        c_carry = cnt[:, _CHUNK - 1]
        bit = jnp.left_shift(jnp.bitwise_and(
            jax.lax.bitcast_convert_type(cnt + jnp.float32(8388608.0),
                                         jnp.int32), 1), 31)
        e_bits = jax.lax.bitcast_convert_type(jnp.exp2(s), jnp.int32)
        o_ref[:, sl] = jax.lax.bitcast_convert_type(e_bits | bit, jnp.float32)


def kernel(x):
    m, n = x.shape
    block_rows = 2048
    u = jnp.triu(jnp.ones((_CHUNK, _CHUNK), jnp.bfloat16))
    return pl.pallas_call(
        _cumprod_block,
        out_shape=jax.ShapeDtypeStruct((m, n), x.dtype),
        grid=(m // block_rows,),
        in_specs=[
            pl.BlockSpec((block_rows, n), lambda i: (i, 0)),
            pl.BlockSpec((_CHUNK, _CHUNK), lambda i: (0, 0)),
        ],
        out_specs=pl.BlockSpec((block_rows, n), lambda i: (i, 0)),
    )(x, u)
